# Initial kernel scaffold; baseline (speedup 1.0000x reference)
import functools

import jax
import jax.numpy as jnp
from jax import lax
from jax.experimental import pallas as pl
from jax.experimental.pallas import tpu as pltpu
from jax.experimental.pallas import tpu_sc as plsc

N = 10000
N_PAD = 10240
E = 320000
D = 128

NC = 2
NS = 16
NW = NC * NS
EW = E // NW
K = 125
CH = EW // K
ZR = 128
RPS = N_PAD // NS

_sc_mesh = plsc.VectorSubcoreMesh(core_axis_name="c", subcore_axis_name="s")


def _deg_body(src_hbm, dst_hbm, out_hbm, sidx, didx, deg):
    c = lax.axis_index("c")
    s = lax.axis_index("s")
    pltpu.sync_copy(src_hbm.at[c, s], sidx)
    pltpu.sync_copy(dst_hbm.at[c, s], didx)

    @pl.loop(0, (N_PAD * 2) // 16)
    def _zero(i):
        deg[pl.ds(i * 16, 16)] = jnp.zeros((16,), jnp.float32)

    ones = jnp.ones((16,), jnp.float32)

    @pl.loop(0, EW // 16)
    def _hist(i):
        sv = sidx[pl.ds(i * 16, 16)]
        dv = didx[pl.ds(i * 16, 16)]
        plsc.addupdate_scatter(deg, [sv * 2], ones)
        plsc.addupdate_scatter(deg, [dv * 2 + 1], ones)

    pltpu.sync_copy(deg, out_hbm.at[c, s])


_deg_kernel = functools.partial(
    pl.kernel,
    out_type=jax.ShapeDtypeStruct((NC, NS, N_PAD * 2), jnp.float32),
    mesh=_sc_mesh,
    scratch_types=[
        pltpu.VMEM((EW,), jnp.int32),
        pltpu.VMEM((EW,), jnp.int32),
        pltpu.VMEM((N_PAD * 2,), jnp.float32),
    ],
)(_deg_body)


def _agg_body(xs_hbm, src_hbm, dst_hbm, out_hbm, sidx, didx, rows0, rows1,
              zbuf, acc, gsem):
    c = lax.axis_index("c")
    s = lax.axis_index("s")
    pltpu.sync_copy(src_hbm.at[c, s], sidx)
    pltpu.sync_copy(dst_hbm.at[c, s], didx)

    @pl.loop(0, ZR)
    def _zfill(r):
        for l in range(D // 16):
            zbuf[r, pl.ds(l * 16, 16)] = jnp.zeros((16,), jnp.float32)

    @pl.loop(0, RPS // ZR)
    def _zacc(t):
        pltpu.sync_copy(zbuf, acc.at[pl.ds(s * RPS + t * ZR, ZR)])

    plsc.subcore_barrier()

    @pl.loop(0, CH, step=2)
    def _edges(j):
        cp0 = pltpu.async_copy(xs_hbm.at[sidx.at[j]], rows0, gsem)
        cp1 = pltpu.async_copy(xs_hbm.at[sidx.at[j + 1]], rows1, gsem)
        cp0.wait()
        pltpu.sync_copy(rows0, acc.at[didx.at[j]], add=True)
        cp1.wait()
        pltpu.sync_copy(rows1, acc.at[didx.at[j + 1]], add=True)

    plsc.subcore_barrier()
    pltpu.sync_copy(acc.at[pl.ds(s * RPS, RPS)],
                    out_hbm.at[c, pl.ds(s * RPS, RPS)])


_agg_kernel = functools.partial(
    pl.kernel,
    out_type=jax.ShapeDtypeStruct((NC, N_PAD, D), jnp.float32),
    mesh=_sc_mesh,
    scratch_types=[
        pltpu.VMEM((CH, K), jnp.int32),
        pltpu.VMEM((CH, K), jnp.int32),
        pltpu.VMEM((K, D), jnp.float32),
        pltpu.VMEM((K, D), jnp.float32),
        pltpu.VMEM((ZR, D), jnp.float32),
        pltpu.VMEM_SHARED((N_PAD, D), jnp.float32),
        pltpu.SemaphoreType.DMA,
    ],
)(_agg_body)


def _norm_body(dp_ref, x_ref, xs_ref, nrm_ref):
    d = jnp.sum(dp_ref[...], axis=0)
    nrm = lax.rsqrt(jnp.maximum(d, 1.0))
    nrm_ref[...] = nrm
    xs_ref[...] = x_ref[...] * nrm[:, 0:1]


def _norm_call(dp, x, block):
    grid = (N_PAD // block,)
    return pl.pallas_call(
        _norm_body,
        grid=grid,
        in_specs=[
            pl.BlockSpec((NW, block, 2), lambda i: (0, i, 0)),
            pl.BlockSpec((block, D), lambda i: (i, 0)),
        ],
        out_specs=[
            pl.BlockSpec((block, D), lambda i: (i, 0)),
            pl.BlockSpec((block, 2), lambda i: (i, 0)),
        ],
        out_shape=[
            jax.ShapeDtypeStruct((N_PAD, D), jnp.float32),
            jax.ShapeDtypeStruct((N_PAD, 2), jnp.float32),
        ],
    )(dp, x)


def _mm_body(p_ref, nrm_ref, w_ref, b_ref, o_ref, *, layer1):
    agg = p_ref[0] + p_ref[1]
    nrm = nrm_ref[...]
    y = jnp.dot(agg * nrm[:, 1:2], w_ref[...],
                preferred_element_type=jnp.float32) + b_ref[...]
    if layer1:
        y = jnp.maximum(y, 0.0) * nrm[:, 0:1]
    o_ref[...] = y


def _mm_call(p, nrm, w, b, layer1, block):
    grid = (N_PAD // block,)
    return pl.pallas_call(
        functools.partial(_mm_body, layer1=layer1),
        grid=grid,
        in_specs=[
            pl.BlockSpec((NC, block, D), lambda i: (0, i, 0)),
            pl.BlockSpec((block, 2), lambda i: (i, 0)),
            pl.BlockSpec((D, D), lambda i: (0, 0)),
            pl.BlockSpec((1, D), lambda i: (0, 0)),
        ],
        out_specs=pl.BlockSpec((block, D), lambda i: (i, 0)),
        out_shape=jax.ShapeDtypeStruct((N_PAD, D), jnp.float32),
    )(p, nrm, w, b)


@jax.jit
def kernel(h, edge_index, W1, b1, W2, b2):
    src = edge_index[0].reshape(NC, NS, CH, K)
    dst = edge_index[1].reshape(NC, NS, CH, K)
    srcf = edge_index[0].reshape(NC, NS, EW)
    dstf = edge_index[1].reshape(NC, NS, EW)

    x = jnp.pad(jnp.transpose(h, (1, 0)), ((0, N_PAD - N), (0, 0)))

    degp = _deg_kernel(srcf, dstf).reshape(NW, N_PAD, 2)
    xs1, nrm = _norm_call(degp, x, block=1280)

    p1 = _agg_kernel(xs1, src, dst)
    xs2 = _mm_call(p1, nrm, W1, b1.reshape(1, D), True, block=1280)

    p2 = _agg_kernel(xs2, src, dst)
    out = _mm_call(p2, nrm, W2, b2.reshape(1, D), False, block=1280)

    return jnp.transpose(out[:N], (1, 0))


# speedup vs baseline: 6.2690x; 6.2690x over previous
import functools

import jax
import jax.numpy as jnp
from jax import lax
from jax.experimental import pallas as pl
from jax.experimental.pallas import tpu as pltpu
from jax.experimental.pallas import tpu_sc as plsc

N = 10000
N_PAD = 10240
E = 320000
D = 128
DH = D // 2

NC = 2
NS = 16
NW = NC * NS
EW = E // NW
ES = E // NS
K = 125
CH = ES // K
ZR = 128
RPS = N_PAD // NS

_sc_mesh = plsc.VectorSubcoreMesh(core_axis_name="c", subcore_axis_name="s")
_sc_params = pltpu.CompilerParams(needs_layout_passes=False,
                                  use_tc_tiling_on_sc=False)


def _deg_body(src_hbm, dst_hbm, out_hbm, sidx, didx, deg):
    c = lax.axis_index("c")
    s = lax.axis_index("s")
    pltpu.sync_copy(src_hbm.at[c, s], sidx)
    pltpu.sync_copy(dst_hbm.at[c, s], didx)

    @pl.loop(0, (N_PAD * 2) // 16)
    def _zero(i):
        deg[pl.ds(i * 16, 16)] = jnp.zeros((16,), jnp.float32)

    ones = jnp.ones((16,), jnp.float32)

    @pl.loop(0, EW // 16)
    def _hist(i):
        sv = sidx[pl.ds(i * 16, 16)]
        dv = didx[pl.ds(i * 16, 16)]
        plsc.addupdate_scatter(deg, [sv * 2], ones)
        plsc.addupdate_scatter(deg, [dv * 2 + 1], ones)

    pltpu.sync_copy(deg, out_hbm.at[c, s])


_deg_kernel = functools.partial(
    pl.kernel,
    out_type=jax.ShapeDtypeStruct((NC, NS, N_PAD * 2), jnp.float32),
    mesh=_sc_mesh,
    scratch_types=[
        pltpu.VMEM((EW,), jnp.int32),
        pltpu.VMEM((EW,), jnp.int32),
        pltpu.VMEM((N_PAD * 2,), jnp.float32),
    ],
    compiler_params=_sc_params,
)(_deg_body)


def _agg_body(xs_hbm, src_hbm, dst_hbm, out_hbm, sidx, didx, rows0, rows1,
              zbuf, acc, gsem):
    c = lax.axis_index("c")
    s = lax.axis_index("s")
    pltpu.sync_copy(src_hbm.at[s], sidx)
    pltpu.sync_copy(dst_hbm.at[s], didx)

    @pl.loop(0, ZR)
    def _zfill(r):
        for l in range(DH // 16):
            zbuf[r, pl.ds(l * 16, 16)] = jnp.zeros((16,), jnp.float32)

    @pl.loop(0, RPS // ZR)
    def _zacc(t):
        pltpu.sync_copy(zbuf, acc.at[pl.ds(s * RPS + t * ZR, ZR)])

    plsc.subcore_barrier()

    @pl.loop(0, CH, step=2)
    def _edges(j):
        cp0 = pltpu.async_copy(xs_hbm.at[c].at[sidx.at[j]], rows0, gsem)
        cp1 = pltpu.async_copy(xs_hbm.at[c].at[sidx.at[j + 1]], rows1, gsem)
        cp0.wait()
        pltpu.sync_copy(rows0, acc.at[didx.at[j]], add=True)
        cp1.wait()
        pltpu.sync_copy(rows1, acc.at[didx.at[j + 1]], add=True)

    plsc.subcore_barrier()
    pltpu.sync_copy(acc.at[pl.ds(s * RPS, RPS)],
                    out_hbm.at[c, pl.ds(s * RPS, RPS)])


_agg_kernel = functools.partial(
    pl.kernel,
    out_type=jax.ShapeDtypeStruct((NC, N_PAD, DH), jnp.float32),
    mesh=_sc_mesh,
    scratch_types=[
        pltpu.VMEM((CH, K), jnp.int32),
        pltpu.VMEM((CH, K), jnp.int32),
        pltpu.VMEM((K, DH), jnp.float32),
        pltpu.VMEM((K, DH), jnp.float32),
        pltpu.VMEM((ZR, DH), jnp.float32),
        pltpu.VMEM_SHARED((N_PAD, DH), jnp.float32),
        pltpu.SemaphoreType.DMA,
    ],
    compiler_params=_sc_params,
)(_agg_body)


def _norm_body(dp_ref, x_ref, xs_ref, nrm_ref):
    d = jnp.sum(dp_ref[...], axis=0)
    nrm = lax.rsqrt(jnp.maximum(d, 1.0))
    nrm_ref[...] = nrm
    xs = x_ref[...] * nrm[:, 0:1]
    xs_ref[0] = xs[:, :DH]
    xs_ref[1] = xs[:, DH:]


def _norm_call(dp, x, block):
    grid = (N_PAD // block,)
    return pl.pallas_call(
        _norm_body,
        grid=grid,
        in_specs=[
            pl.BlockSpec((NW, block, 2), lambda i: (0, i, 0)),
            pl.BlockSpec((block, D), lambda i: (i, 0)),
        ],
        out_specs=[
            pl.BlockSpec((NC, block, DH), lambda i: (0, i, 0)),
            pl.BlockSpec((block, 2), lambda i: (i, 0)),
        ],
        out_shape=[
            jax.ShapeDtypeStruct((NC, N_PAD, DH), jnp.float32),
            jax.ShapeDtypeStruct((N_PAD, 2), jnp.float32),
        ],
    )(dp, x)


def _mm_body(p_ref, nrm_ref, w_ref, b_ref, o_ref, *, layer1):
    nrm = nrm_ref[...]
    nd = nrm[:, 1:2]
    y = (jnp.dot(p_ref[0] * nd, w_ref[:DH, :],
                 preferred_element_type=jnp.float32)
         + jnp.dot(p_ref[1] * nd, w_ref[DH:, :],
                   preferred_element_type=jnp.float32)
         + b_ref[...])
    if layer1:
        y = jnp.maximum(y, 0.0) * nrm[:, 0:1]
        o_ref[0] = y[:, :DH]
        o_ref[1] = y[:, DH:]
    else:
        o_ref[...] = y


def _mm_call(p, nrm, w, b, layer1, block):
    grid = (N_PAD // block,)
    if layer1:
        out_spec = pl.BlockSpec((NC, block, DH), lambda i: (0, i, 0))
        out_shape = jax.ShapeDtypeStruct((NC, N_PAD, DH), jnp.float32)
    else:
        out_spec = pl.BlockSpec((block, D), lambda i: (i, 0))
        out_shape = jax.ShapeDtypeStruct((N_PAD, D), jnp.float32)
    return pl.pallas_call(
        functools.partial(_mm_body, layer1=layer1),
        grid=grid,
        in_specs=[
            pl.BlockSpec((NC, block, DH), lambda i: (0, i, 0)),
            pl.BlockSpec((block, 2), lambda i: (i, 0)),
            pl.BlockSpec((D, D), lambda i: (0, 0)),
            pl.BlockSpec((1, D), lambda i: (0, 0)),
        ],
        out_specs=out_spec,
        out_shape=out_shape,
    )(p, nrm, w, b)


@jax.jit
def kernel(h, edge_index, W1, b1, W2, b2):
    src = edge_index[0].reshape(NS, CH, K)
    dst = edge_index[1].reshape(NS, CH, K)
    srcf = edge_index[0].reshape(NC, NS, EW)
    dstf = edge_index[1].reshape(NC, NS, EW)

    x = jnp.pad(jnp.transpose(h, (1, 0)), ((0, N_PAD - N), (0, 0)))

    degp = _deg_kernel(srcf, dstf).reshape(NW, N_PAD, 2)
    xs1, nrm = _norm_call(degp, x, block=1280)

    p1 = _agg_kernel(xs1, src, dst)
    xs2 = _mm_call(p1, nrm, W1, b1.reshape(1, D), True, block=1280)

    p2 = _agg_kernel(xs2, src, dst)
    out = _mm_call(p2, nrm, W2, b2.reshape(1, D), False, block=1280)

    return jnp.transpose(out[:N], (1, 0))


# speedup vs baseline: 12.0451x; 1.9214x over previous
import functools

import jax
import jax.numpy as jnp
from jax import lax
from jax.experimental import pallas as pl
from jax.experimental.pallas import tpu as pltpu
from jax.experimental.pallas import tpu_sc as plsc

N = 10000
N_PAD = 10240
E = 320000
D = 128
DH = D // 2

NC = 2
NS = 16
NW = NC * NS
EW = E // NW
ES = E // NS
K = 125
CH = ES // K
ZR = 128
RPS = N_PAD // NS

_sc_mesh = plsc.VectorSubcoreMesh(core_axis_name="c", subcore_axis_name="s")
_sc_params = pltpu.CompilerParams(needs_layout_passes=False,
                                  use_tc_tiling_on_sc=False)


def _deg_body(src_hbm, dst_hbm, out_hbm, sidx, didx, dego, degi):
    c = lax.axis_index("c")
    s = lax.axis_index("s")
    pltpu.sync_copy(src_hbm.at[c, s], sidx)
    pltpu.sync_copy(dst_hbm.at[c, s], didx)

    @pl.loop(0, N_PAD // 16)
    def _zero(i):
        dego[pl.ds(i * 16, 16)] = jnp.zeros((16,), jnp.float32)
        degi[pl.ds(i * 16, 16)] = jnp.zeros((16,), jnp.float32)

    ones = jnp.ones((16,), jnp.float32)

    @pl.loop(0, EW // 16)
    def _hist(i):
        sv = sidx[pl.ds(i * 16, 16)]
        dv = didx[pl.ds(i * 16, 16)]
        plsc.addupdate_scatter(dego, [sv], ones)
        plsc.addupdate_scatter(degi, [dv], ones)

    pltpu.sync_copy(dego, out_hbm.at[c, s, 0])
    pltpu.sync_copy(degi, out_hbm.at[c, s, 1])


_deg_kernel = functools.partial(
    pl.kernel,
    out_type=jax.ShapeDtypeStruct((NC, NS, 2, N_PAD), jnp.float32),
    mesh=_sc_mesh,
    scratch_types=[
        pltpu.VMEM((EW,), jnp.int32),
        pltpu.VMEM((EW,), jnp.int32),
        pltpu.VMEM((N_PAD,), jnp.float32),
        pltpu.VMEM((N_PAD,), jnp.float32),
    ],
    compiler_params=_sc_params,
)(_deg_body)


def _agg_body(xs_hbm, src_hbm, dst_hbm, out_hbm, sidx, didx, b0, b1, b2, b3,
              zbuf, acc, gsem, ssem):
    c = lax.axis_index("c")
    s = lax.axis_index("s")
    pltpu.sync_copy(src_hbm.at[s], sidx)
    pltpu.sync_copy(dst_hbm.at[s], didx)

    @pl.loop(0, ZR)
    def _zfill(r):
        for l in range(DH // 16):
            zbuf[r, pl.ds(l * 16, 16)] = jnp.zeros((16,), jnp.float32)

    @pl.loop(0, RPS // ZR)
    def _zacc(t):
        pltpu.sync_copy(zbuf, acc.at[pl.ds(s * RPS + t * ZR, ZR)])

    plsc.subcore_barrier()

    bufs = [b0, b1, b2, b3]
    xsc = xs_hbm.at[c]
    pltpu.async_copy(xsc.at[sidx.at[0]], b0, gsem)
    pltpu.async_copy(xsc.at[sidx.at[1]], b1, gsem)

    @pl.loop(0, CH, step=4)
    def _edges(j):
        for k in range(4):
            p = j + k
            buf = bufs[k]
            nbuf = bufs[(k + 2) % 4]
            pltpu.make_async_copy(xsc.at[sidx.at[p]], buf, gsem).wait()
            pltpu.async_copy(buf, acc.at[didx.at[p]], ssem, add=True)

            @pl.when(p >= 2)
            def _wait_scatter():
                pltpu.make_async_copy(nbuf, acc.at[didx.at[p]], ssem).wait()

            @pl.when(p + 2 < CH)
            def _next_gather():
                pltpu.async_copy(xsc.at[sidx.at[p + 2]], nbuf, gsem)

    for k in range(2):
        pltpu.make_async_copy(bufs[k], acc.at[didx.at[k]], ssem).wait()

    plsc.subcore_barrier()
    pltpu.sync_copy(acc.at[pl.ds(s * RPS, RPS)],
                    out_hbm.at[c, pl.ds(s * RPS, RPS)])


_agg_kernel = functools.partial(
    pl.kernel,
    out_type=jax.ShapeDtypeStruct((NC, N_PAD, DH), jnp.float32),
    mesh=_sc_mesh,
    scratch_types=[
        pltpu.VMEM((CH, K), jnp.int32),
        pltpu.VMEM((CH, K), jnp.int32),
        pltpu.VMEM((K, DH), jnp.float32),
        pltpu.VMEM((K, DH), jnp.float32),
        pltpu.VMEM((K, DH), jnp.float32),
        pltpu.VMEM((K, DH), jnp.float32),
        pltpu.VMEM((ZR, DH), jnp.float32),
        pltpu.VMEM_SHARED((N_PAD, DH), jnp.float32),
        pltpu.SemaphoreType.DMA,
        pltpu.SemaphoreType.DMA,
    ],
    compiler_params=_sc_params,
)(_agg_body)


def _norm_body(dp_ref, h_ref, xs_ref, nrm_ref):
    d = jnp.sum(dp_ref[...], axis=0)
    nrm = lax.rsqrt(jnp.maximum(d, 1.0))
    nrm_ref[...] = nrm.T
    xs = jnp.transpose(h_ref[...] * nrm[0:1, :], (1, 0))
    xs_ref[0] = xs[:, :DH]
    xs_ref[1] = xs[:, DH:]


def _norm_call(dp, hp, block):
    grid = (N_PAD // block,)
    return pl.pallas_call(
        _norm_body,
        grid=grid,
        in_specs=[
            pl.BlockSpec((NW, 2, block), lambda i: (0, 0, i)),
            pl.BlockSpec((D, block), lambda i: (0, i)),
        ],
        out_specs=[
            pl.BlockSpec((NC, block, DH), lambda i: (0, i, 0)),
            pl.BlockSpec((block, 2), lambda i: (i, 0)),
        ],
        out_shape=[
            jax.ShapeDtypeStruct((NC, N_PAD, DH), jnp.float32),
            jax.ShapeDtypeStruct((N_PAD, 2), jnp.float32),
        ],
    )(dp, hp)


def _mm_body(p_ref, nrm_ref, w_ref, b_ref, o_ref, *, layer1):
    nrm = nrm_ref[...]
    nd = nrm[:, 1:2]
    y = (jnp.dot(p_ref[0] * nd, w_ref[:DH, :],
                 preferred_element_type=jnp.float32)
         + jnp.dot(p_ref[1] * nd, w_ref[DH:, :],
                   preferred_element_type=jnp.float32)
         + b_ref[...])
    if layer1:
        y = jnp.maximum(y, 0.0) * nrm[:, 0:1]
        o_ref[0] = y[:, :DH]
        o_ref[1] = y[:, DH:]
    else:
        o_ref[...] = y


def _mm_call(p, nrm, w, b, layer1, block):
    grid = (N_PAD // block,)
    if layer1:
        out_spec = pl.BlockSpec((NC, block, DH), lambda i: (0, i, 0))
        out_shape = jax.ShapeDtypeStruct((NC, N_PAD, DH), jnp.float32)
    else:
        out_spec = pl.BlockSpec((block, D), lambda i: (i, 0))
        out_shape = jax.ShapeDtypeStruct((N_PAD, D), jnp.float32)
    return pl.pallas_call(
        functools.partial(_mm_body, layer1=layer1),
        grid=grid,
        in_specs=[
            pl.BlockSpec((NC, block, DH), lambda i: (0, i, 0)),
            pl.BlockSpec((block, 2), lambda i: (i, 0)),
            pl.BlockSpec((D, D), lambda i: (0, 0)),
            pl.BlockSpec((1, D), lambda i: (0, 0)),
        ],
        out_specs=out_spec,
        out_shape=out_shape,
    )(p, nrm, w, b)


@jax.jit
def kernel(h, edge_index, W1, b1, W2, b2):
    src = edge_index[0].reshape(NS, CH, K)
    dst = edge_index[1].reshape(NS, CH, K)
    srcf = edge_index[0].reshape(NC, NS, EW)
    dstf = edge_index[1].reshape(NC, NS, EW)

    hp = jnp.pad(h, ((0, 0), (0, N_PAD - N)))

    degp = _deg_kernel(srcf, dstf).reshape(NW, 2, N_PAD)
    xs1, nrm = _norm_call(degp, hp, block=1280)

    p1 = _agg_kernel(xs1, src, dst)
    xs2 = _mm_call(p1, nrm, W1, b1.reshape(1, D), True, block=1280)

    p2 = _agg_kernel(xs2, src, dst)
    out = _mm_call(p2, nrm, W2, b2.reshape(1, D), False, block=1280)

    return jnp.transpose(out[:N], (1, 0))
